# baseline (speedup 1.0000x reference)
import jax
import jax.numpy as jnp
from jax import lax
from jax.experimental import pallas as pl
from jax.experimental.pallas import tpu as pltpu
from jax.experimental.pallas import tpu_sc as plsc

_WEIGHT = 0.7
_L = 16


def _mse_alive_body(data_hbm, out_hbm, data_v, out_v):
    @pl.when(jnp.logical_and(lax.axis_index("c") == 0, lax.axis_index("s") == 0))
    def _():
        pltpu.sync_copy(data_hbm, data_v)
        v = data_v[...]
        x0, x1 = v[0], v[1]
        t0, t1 = v[2], v[3]
        a0, a1 = v[4], v[5]
        p0, p1 = v[6], v[7]

        sq0 = (x0 - t0) * (x0 - t0)
        sq1 = (x1 - t1) * (x1 - t1)
        cv0 = (p0 == 2.0) & ((x0 < t0) | (a0 == 0.0))
        cv1 = (p1 == 2.0) & ((x1 < t1) | (a1 == 0.0))
        cp0 = p0 == 1.0
        cp1 = p1 == 1.0

        one = jnp.float32(1.0)
        zero = jnp.float32(0.0)
        valid_count = jnp.where(cv0, one, zero) + jnp.where(cv1, one, zero)
        valid_sum = jnp.where(cv0, sq0, zero) + jnp.where(cv1, sq1, zero)
        pseudo_count = jnp.where(cp0, one, zero) + jnp.where(cp1, one, zero)
        pseudo_sum = jnp.where(cp0, sq0, zero) + jnp.where(cp1, sq1, zero)

        num_true = jnp.where(valid_count > zero, valid_sum * _WEIGHT, zero)
        num_pseudo = jnp.where(pseudo_count > zero, pseudo_sum * (1.0 - _WEIGHT), zero)
        lt = jnp.broadcast_to(num_true, (_L,)) / jnp.broadcast_to(
            jnp.maximum(valid_count, one), (_L,)
        )
        lp = jnp.broadcast_to(num_pseudo, (_L,)) / jnp.broadcast_to(
            jnp.maximum(pseudo_count, one), (_L,)
        )
        out_v[...] = lt + lp
        pltpu.sync_copy(out_v, out_hbm)


def kernel(inputs, target, target_label, alive, pseudo, bins):
    x = jnp.reshape(inputs, (-1,))[:2].astype(jnp.float32)
    t = jnp.reshape(target, (-1,))[:2].astype(jnp.float32)
    a = jnp.reshape(alive, (-1,))[:2].astype(jnp.float32)
    p = jnp.reshape(pseudo, (-1,))[:2].astype(jnp.float32)

    data = jnp.pad(jnp.concatenate([x, t, a, p]), (0, _L - 8))

    run = pl.kernel(
        _mse_alive_body,
        mesh=plsc.VectorSubcoreMesh(core_axis_name="c", subcore_axis_name="s"),
        out_type=jax.ShapeDtypeStruct((_L,), jnp.float32),
        scratch_types=[
            pltpu.VMEM((_L,), jnp.float32),
            pltpu.VMEM((_L,), jnp.float32),
        ],
    )
    out = run(data)
    return out[0]


# speedup vs baseline: 1.0246x; 1.0246x over previous
import jax
import jax.numpy as jnp
from jax import lax
from jax.experimental import pallas as pl
from jax.experimental.pallas import tpu as pltpu
from jax.experimental.pallas import tpu_sc as plsc

_WEIGHT = 0.7
_L = 16


def _mse_alive_body(x_hbm, t_hbm, a_hbm, p_hbm, out_hbm, f_v, i_v, out_v):
    @pl.when(jnp.logical_and(lax.axis_index("c") == 0, lax.axis_index("s") == 0))
    def _():
        pltpu.sync_copy(x_hbm, f_v.at[pl.ds(0, 2)])
        pltpu.sync_copy(t_hbm, f_v.at[pl.ds(8, 2)])
        pltpu.sync_copy(a_hbm, i_v.at[pl.ds(0, 2)])
        pltpu.sync_copy(p_hbm, i_v.at[pl.ds(8, 2)])
        f = f_v[...]
        iv = i_v[...]
        x0, x1, t0, t1 = f[0], f[1], f[8], f[9]
        a0, a1, p0, p1 = iv[0], iv[1], iv[8], iv[9]

        sq0 = (x0 - t0) * (x0 - t0)
        sq1 = (x1 - t1) * (x1 - t1)
        cv0 = (p0 == 2) & ((x0 < t0) | (a0 == 0))
        cv1 = (p1 == 2) & ((x1 < t1) | (a1 == 0))
        cp0 = p0 == 1
        cp1 = p1 == 1

        one = jnp.float32(1.0)
        zero = jnp.float32(0.0)
        half = jnp.float32(0.5)
        valid_count = jnp.where(cv0, one, zero) + jnp.where(cv1, one, zero)
        valid_sum = jnp.where(cv0, sq0, zero) + jnp.where(cv1, sq1, zero)
        pseudo_count = jnp.where(cp0, one, zero) + jnp.where(cp1, one, zero)
        pseudo_sum = jnp.where(cp0, sq0, zero) + jnp.where(cp1, sq1, zero)

        loss_true = jnp.where(
            valid_count > zero,
            valid_sum * jnp.where(valid_count == 2.0, half, one),
            zero,
        )
        loss_pseudo = jnp.where(
            pseudo_count > zero,
            pseudo_sum * jnp.where(pseudo_count == 2.0, half, one),
            zero,
        )
        loss = loss_true * _WEIGHT + loss_pseudo * (1.0 - _WEIGHT)

        out_v[...] = jnp.broadcast_to(loss, (_L,))
        pltpu.sync_copy(out_v, out_hbm)


def kernel(inputs, target, target_label, alive, pseudo, bins):
    run = pl.kernel(
        _mse_alive_body,
        mesh=plsc.VectorSubcoreMesh(
            core_axis_name="c", subcore_axis_name="s", num_cores=1
        ),
        out_type=jax.ShapeDtypeStruct((_L,), jnp.float32),
        scratch_types=[
            pltpu.VMEM((_L,), jnp.float32),
            pltpu.VMEM((_L,), jnp.int32),
            pltpu.VMEM((_L,), jnp.float32),
        ],
    )
    out = run(inputs, target, alive, pseudo)
    return out[0]


# speedup vs baseline: 1.1618x; 1.1339x over previous
import jax
import jax.numpy as jnp
from jax import lax
from jax.experimental import pallas as pl
from jax.experimental.pallas import tpu as pltpu
from jax.experimental.pallas import tpu_sc as plsc

_WEIGHT = 0.7
_L = 16


def _mse_alive_body(data_hbm, out_hbm, d_sp, d_s, o_s, o_sp):
    @pl.when(lax.axis_index("c") == 0)
    def _():
        pltpu.sync_copy(data_hbm, d_sp)
        pltpu.sync_copy(d_sp, d_s)
        x0, x1, t0, t1 = d_s[0], d_s[1], d_s[2], d_s[3]
        a0, a1, p0, p1 = d_s[4], d_s[5], d_s[6], d_s[7]

        sq0 = (x0 - t0) * (x0 - t0)
        sq1 = (x1 - t1) * (x1 - t1)
        cv0 = (p0 == 2.0) & ((x0 < t0) | (a0 == 0.0))
        cv1 = (p1 == 2.0) & ((x1 < t1) | (a1 == 0.0))
        cp0 = p0 == 1.0
        cp1 = p1 == 1.0

        one = jnp.float32(1.0)
        zero = jnp.float32(0.0)
        half = jnp.float32(0.5)
        valid_count = jnp.where(cv0, one, zero) + jnp.where(cv1, one, zero)
        valid_sum = jnp.where(cv0, sq0, zero) + jnp.where(cv1, sq1, zero)
        pseudo_count = jnp.where(cp0, one, zero) + jnp.where(cp1, one, zero)
        pseudo_sum = jnp.where(cp0, sq0, zero) + jnp.where(cp1, sq1, zero)

        loss_true = jnp.where(
            valid_count > zero,
            valid_sum * jnp.where(valid_count == 2.0, half, one),
            zero,
        )
        loss_pseudo = jnp.where(
            pseudo_count > zero,
            pseudo_sum * jnp.where(pseudo_count == 2.0, half, one),
            zero,
        )
        loss = loss_true * _WEIGHT + loss_pseudo * (1.0 - _WEIGHT)

        o_s[0] = loss
        pltpu.sync_copy(o_s, out_hbm)


def kernel(inputs, target, target_label, alive, pseudo, bins):
    x = jnp.reshape(inputs, (-1,))[:2]
    t = target
    a = alive.astype(jnp.float32)
    p = pseudo.astype(jnp.float32)
    data = jnp.pad(jnp.concatenate([x, t, a, p]), (0, _L - 8))

    run = pl.kernel(
        _mse_alive_body,
        mesh=plsc.ScalarSubcoreMesh(axis_name="c", num_cores=1),
        out_type=jax.ShapeDtypeStruct((_L,), jnp.float32),
        scratch_types=[
            pltpu.VMEM_SHARED((_L,), jnp.float32),
            pltpu.SMEM((_L,), jnp.float32),
            pltpu.SMEM((_L,), jnp.float32),
            pltpu.VMEM_SHARED((_L,), jnp.float32),
        ],
    )
    out = run(data)
    return out[0]
